# baseline (speedup 1.0000x reference)
import functools

import jax
import jax.numpy as jnp
from jax import lax
from jax.experimental import pallas as pl
from jax.experimental.pallas import tpu as pltpu
from jax.experimental.pallas import tpu_sc as plsc

N_USERS = 1000000
N_ITEMS = 1000000
N_FACTORS = 32
BATCH = 16384

NUM_CORES = 2
NUM_SUBCORES = 16
LANES = 16
NUM_WORKERS = NUM_CORES * NUM_SUBCORES
B_PER_W = BATCH // NUM_WORKERS
CHUNK = 128
N_CHUNKS = B_PER_W // CHUNK
BLOCKS = B_PER_W // LANES


@functools.partial(
    pl.kernel,
    mesh=plsc.VectorSubcoreMesh(core_axis_name="c", subcore_axis_name="s"),
    out_type=jax.ShapeDtypeStruct((BATCH,), jnp.float32),
    compiler_params=pltpu.CompilerParams(
        needs_layout_passes=False, use_tc_tiling_on_sc=False),
    scratch_types=[
        pltpu.VMEM((N_CHUNKS, CHUNK), jnp.int32),
        pltpu.VMEM((N_CHUNKS, CHUNK), jnp.int32),
        pltpu.VMEM((B_PER_W, N_FACTORS), jnp.float32),
        pltpu.VMEM((B_PER_W, N_FACTORS), jnp.float32),
        pltpu.VMEM((B_PER_W,), jnp.float32),
        pltpu.VMEM((B_PER_W,), jnp.float32),
        pltpu.VMEM((B_PER_W,), jnp.float32),
        pltpu.SemaphoreType.DMA,
    ],
)
def _sc_kernel(user_t, item_t, ubias_t, ibias_t, uid, iid, out,
               uidx, iidx, urows, irows, ubias, ibias, outv, sem):
    wid = lax.axis_index("s") * NUM_CORES + lax.axis_index("c")

    row0 = wid * N_CHUNKS
    pltpu.sync_copy(uid.at[pl.ds(row0, N_CHUNKS)], uidx)
    pltpu.sync_copy(iid.at[pl.ds(row0, N_CHUNKS)], iidx)

    copies = []
    for j in range(N_CHUNKS):
        copies.append(pltpu.async_copy(
            user_t.at[uidx.at[j]], urows.at[pl.ds(j * CHUNK, CHUNK)], sem))
        copies.append(pltpu.async_copy(
            item_t.at[iidx.at[j]], irows.at[pl.ds(j * CHUNK, CHUNK)], sem))
        copies.append(pltpu.async_copy(
            ubias_t.at[uidx.at[j]], ubias.at[pl.ds(j * CHUNK, CHUNK)], sem))
        copies.append(pltpu.async_copy(
            ibias_t.at[iidx.at[j]], ibias.at[pl.ds(j * CHUNK, CHUNK)], sem))
    for c in copies:
        c.wait()

    lane = lax.iota(jnp.int32, LANES)

    def block_body(bb, _):
        o0 = bb * LANES
        acc = ubias[pl.ds(o0, LANES)] + ibias[pl.ds(o0, LANES)]
        for k in range(LANES):
            r = o0 + k
            prod = (urows[r, pl.ds(0, LANES)] * irows[r, pl.ds(0, LANES)]
                    + urows[r, pl.ds(LANES, LANES)] * irows[r, pl.ds(LANES, LANES)])
            acc = jnp.where(lane == k, jnp.sum(prod), acc)
        outv[pl.ds(o0, LANES)] = acc
        return 0

    lax.fori_loop(0, BLOCKS, block_body, 0)
    pltpu.sync_copy(outv, out.at[pl.ds(wid * B_PER_W, B_PER_W)])


def kernel(user_table, item_table, user_bias_table, item_bias_table,
           user_id, item_id):
    uid2d = user_id.astype(jnp.int32).reshape(BATCH // CHUNK, CHUNK)
    iid2d = item_id.astype(jnp.int32).reshape(BATCH // CHUNK, CHUNK)
    out = _sc_kernel(user_table, item_table,
                     user_bias_table.reshape(-1), item_bias_table.reshape(-1),
                     uid2d, iid2d)
    return out.reshape(BATCH, 1)
